# Initial kernel scaffold; baseline (speedup 1.0000x reference)
import jax
import jax.numpy as jnp
from jax.experimental import pallas as pl


def kernel(x, W, b, alpha, beta):
    raise NotImplementedError("write your pallas kernel here")


# speedup vs baseline: 1.9469x; 1.9469x over previous
import functools

import jax
import jax.numpy as jnp
from jax.experimental import pallas as pl
from jax.experimental.pallas import tpu as pltpu

BLOCK_M = 1024


def _fused_kernel(x_ref, w_ref, b_ref, alpha_ref, beta_ref, o_ref):
    alpha = alpha_ref[:]
    beta = beta_ref[:]
    w_rec = alpha[:, None] * w_ref[:, :] + beta[:, None]
    acc = jax.lax.dot_general(
        x_ref[:, :], w_rec,
        dimension_numbers=(((1,), (1,)), ((), ())),
        preferred_element_type=jnp.float32,
    )
    o_ref[:, :] = jnp.maximum(acc + b_ref[:][None, :], 0.0)


@jax.jit
def kernel(x, W, b, alpha, beta):
    B, S, D = x.shape
    H = W.shape[0]
    M = B * S
    x2 = x.reshape(M, D)

    grid = (M // BLOCK_M,)
    out = pl.pallas_call(
        _fused_kernel,
        grid=grid,
        in_specs=[
            pl.BlockSpec((BLOCK_M, D), lambda i: (i, 0)),
            pl.BlockSpec((H, D), lambda i: (0, 0)),
            pl.BlockSpec((H,), lambda i: (0,)),
            pl.BlockSpec((H,), lambda i: (0,)),
            pl.BlockSpec((H,), lambda i: (0,)),
        ],
        out_specs=pl.BlockSpec((BLOCK_M, H), lambda i: (i, 0)),
        out_shape=jax.ShapeDtypeStruct((M, H), jnp.float32),
        compiler_params=pltpu.CompilerParams(
            dimension_semantics=("arbitrary",),
        ),
    )(x2, W, b, alpha, beta)
    return out.reshape(B, S, H)


# speedup vs baseline: 1.9561x; 1.0047x over previous
import functools

import jax
import jax.numpy as jnp
from jax.experimental import pallas as pl
from jax.experimental.pallas import tpu as pltpu

BLOCK_M = 1024


def _fused_kernel(x_ref, w_ref, b_ref, alpha_ref, beta_ref, o_ref, w_scr):
    @pl.when(pl.program_id(0) == 0)
    def _():
        alpha = alpha_ref[:]
        beta = beta_ref[:]
        w_rec = alpha[:, None] * w_ref[:, :] + beta[:, None]
        w_scr[:, :] = w_rec.astype(jnp.bfloat16)

    acc = jax.lax.dot_general(
        x_ref[:, :].astype(jnp.bfloat16), w_scr[:, :],
        dimension_numbers=(((1,), (1,)), ((), ())),
        preferred_element_type=jnp.float32,
    )
    o_ref[:, :] = jnp.maximum(acc + b_ref[:][None, :], 0.0)


@jax.jit
def kernel(x, W, b, alpha, beta):
    B, S, D = x.shape
    H = W.shape[0]
    M = B * S
    x2 = x.reshape(M, D)

    grid = (M // BLOCK_M,)
    out = pl.pallas_call(
        _fused_kernel,
        grid=grid,
        in_specs=[
            pl.BlockSpec((BLOCK_M, D), lambda i: (i, 0)),
            pl.BlockSpec((H, D), lambda i: (0, 0)),
            pl.BlockSpec((H,), lambda i: (0,)),
            pl.BlockSpec((H,), lambda i: (0,)),
            pl.BlockSpec((H,), lambda i: (0,)),
        ],
        out_specs=pl.BlockSpec((BLOCK_M, H), lambda i: (i, 0)),
        out_shape=jax.ShapeDtypeStruct((M, H), jnp.float32),
        scratch_shapes=[pltpu.VMEM((H, D), jnp.bfloat16)],
        compiler_params=pltpu.CompilerParams(
            dimension_semantics=("arbitrary",),
        ),
    )(x2, W, b, alpha, beta)
    return out.reshape(B, S, H)


# speedup vs baseline: 2.2382x; 1.1442x over previous
import functools

import jax
import jax.numpy as jnp
from jax.experimental import pallas as pl
from jax.experimental.pallas import tpu as pltpu

BLOCK_M = 2048


def _fused_kernel(x_ref, w_ref, b_ref, alpha_ref, beta_ref, o_ref, w_scr):
    @pl.when(pl.program_id(0) == 0)
    def _():
        alpha = alpha_ref[:]
        beta = beta_ref[:]
        w_rec = alpha[:, None] * w_ref[:, :] + beta[:, None]
        w_scr[:, :] = w_rec.astype(jnp.bfloat16)

    acc = jax.lax.dot_general(
        x_ref[:, :].astype(jnp.bfloat16), w_scr[:, :],
        dimension_numbers=(((1,), (1,)), ((), ())),
        preferred_element_type=jnp.float32,
    )
    o_ref[:, :] = jnp.maximum(acc + b_ref[:][None, :], 0.0)


@jax.jit
def kernel(x, W, b, alpha, beta):
    B, S, D = x.shape
    H = W.shape[0]
    M = B * S
    x2 = x.reshape(M, D)

    grid = (M // BLOCK_M,)
    out = pl.pallas_call(
        _fused_kernel,
        grid=grid,
        in_specs=[
            pl.BlockSpec((BLOCK_M, D), lambda i: (i, 0)),
            pl.BlockSpec((H, D), lambda i: (0, 0)),
            pl.BlockSpec((H,), lambda i: (0,)),
            pl.BlockSpec((H,), lambda i: (0,)),
            pl.BlockSpec((H,), lambda i: (0,)),
        ],
        out_specs=pl.BlockSpec((BLOCK_M, H), lambda i: (i, 0)),
        out_shape=jax.ShapeDtypeStruct((M, H), jnp.float32),
        scratch_shapes=[pltpu.VMEM((H, D), jnp.bfloat16)],
        compiler_params=pltpu.CompilerParams(
            dimension_semantics=("arbitrary",),
        ),
    )(x2, W, b, alpha, beta)
    return out.reshape(B, S, H)
